# baseline (speedup 1.0000x reference)
import functools

import jax
import jax.numpy as jnp
from jax import lax
from jax.experimental import pallas as pl
from jax.experimental.pallas import tpu as pltpu
from jax.experimental.pallas import tpu_sc as plsc

N = 10000
E = 320000
NC = 2
NS = 16
NW = NC * NS
CHUNK = 125
NCHUNK = 80

NACC = 10240
ROWS_PER = NACC // NS

D1 = 67
D1P = 80
DHALF = D1P // 2
D2 = 40
D2P = 48


def _sc_degree(dst_hbm, zeros_hbm):
    mesh = plsc.VectorSubcoreMesh(core_axis_name="c", subcore_axis_name="s")

    @functools.partial(
        pl.kernel,
        out_type=jax.ShapeDtypeStruct((NC, NACC), jnp.float32),
        mesh=mesh,
        compiler_params=pltpu.CompilerParams(use_tc_tiling_on_sc=False),
        scratch_types=[
            pltpu.VMEM((NCHUNK, CHUNK), jnp.int32),
            pltpu.VMEM((128,), jnp.float32),
            pltpu.VMEM_SHARED((NACC,), jnp.float32),
        ],
    )
    def deg_kernel(dst_ref, zeros_ref, out_ref, dst_v, ones_v, acc_sh):
        cid = lax.axis_index("c")
        sid = lax.axis_index("s")
        wid = cid * NS + sid

        pltpu.sync_copy(zeros_ref.at[pl.ds(sid * ROWS_PER, ROWS_PER)],
                        acc_sh.at[pl.ds(sid * ROWS_PER, ROWS_PER)])
        pltpu.sync_copy(dst_ref.at[wid], dst_v)
        for i in range(8):
            ones_v[pl.ds(16 * i, 16)] = jnp.ones((16,), jnp.float32)
        plsc.subcore_barrier()

        def body(j, _):
            pltpu.sync_copy(ones_v.at[pl.ds(0, CHUNK)],
                            acc_sh.at[dst_v.at[j]], add=True)
            return ()

        lax.fori_loop(0, NCHUNK, body, (), unroll=False)
        plsc.subcore_barrier()
        pltpu.sync_copy(acc_sh.at[pl.ds(sid * ROWS_PER, ROWS_PER)],
                        out_ref.at[cid].at[pl.ds(sid * ROWS_PER, ROWS_PER)])

    return deg_kernel(dst_hbm, zeros_hbm)


def _sc_aggregate(h_hbm, src_hbm, dst_hbm, zeros_hbm, d, nhalf):
    mesh = plsc.VectorSubcoreMesh(core_axis_name="c", subcore_axis_name="s")

    @functools.partial(
        pl.kernel,
        out_type=jax.ShapeDtypeStruct((nhalf, NC, NACC, d), jnp.float32),
        mesh=mesh,
        compiler_params=pltpu.CompilerParams(use_tc_tiling_on_sc=False),
        scratch_types=[
            pltpu.VMEM((NCHUNK, CHUNK), jnp.int32),
            pltpu.VMEM((NCHUNK, CHUNK), jnp.int32),
            pltpu.VMEM((4, CHUNK, d), jnp.float32),
            pltpu.VMEM_SHARED((NACC, d), jnp.float32),
            pltpu.VMEM_SHARED((NACC, d), jnp.float32),
        ] + [pltpu.SemaphoreType.DMA] * 8,
    )
    def agg_kernel(h_ref, src_ref, dst_ref, zeros_ref, out_ref,
                   src_v, dst_v, rows_v, tbl_sh, acc_sh, *sems):
        cid = lax.axis_index("c")
        sid = lax.axis_index("s")
        wid = cid * NS + sid
        sl = pl.ds(sid * ROWS_PER, ROWS_PER)
        gsem = sems[:4]
        ssem = sems[4:]

        pltpu.sync_copy(src_ref.at[wid], src_v)
        pltpu.sync_copy(dst_ref.at[wid], dst_v)

        for half in range(nhalf):
            pltpu.sync_copy(h_ref.at[sl, pl.ds(half * d, d)], tbl_sh.at[sl])
            pltpu.sync_copy(zeros_ref.at[sl], acc_sh.at[sl])
            plsc.subcore_barrier()

            for k in range(2):
                pltpu.async_copy(tbl_sh.at[src_v.at[k]], rows_v.at[k],
                                 gsem[k])

            @pl.loop(0, NCHUNK, step=4)
            def _(j):
                for b in range(4):
                    k = j + b
                    buf = rows_v.at[b]
                    pltpu.make_async_copy(tbl_sh.at[src_v.at[k]],
                                          buf, gsem[b]).wait()
                    pltpu.async_copy(buf, acc_sh.at[dst_v.at[k]],
                                     ssem[b], add=True)

                    nb = (b + 2) % 4
                    nxt = rows_v.at[nb]

                    @pl.when(jnp.logical_and(k >= 2, k + 2 < NCHUNK))
                    def _():
                        pltpu.make_async_copy(nxt, acc_sh.at[dst_v.at[k]],
                                              ssem[nb]).wait()
                        pltpu.async_copy(tbl_sh.at[src_v.at[k + 2]],
                                         nxt, gsem[nb])

                    @pl.when(k < 2)
                    def _():
                        pltpu.async_copy(tbl_sh.at[src_v.at[k + 2]],
                                         nxt, gsem[nb])

            for b in range(4):
                pltpu.make_async_copy(rows_v.at[b],
                                      acc_sh.at[dst_v.at[0]],
                                      ssem[b]).wait()

            plsc.subcore_barrier()
            pltpu.sync_copy(acc_sh.at[sl],
                            out_ref.at[half].at[cid].at[sl])

    return agg_kernel(h_hbm, src_hbm, dst_hbm, zeros_hbm)


def _dis_from(degs_ref):
    deg = degs_ref[0, :] + degs_ref[1, :] + 1.0
    return lax.rsqrt(deg)[:, None]


def _mm_scale_body(x_ref, w_ref, degs_ref, o_ref):
    h = jnp.dot(x_ref[...], w_ref[...],
                preferred_element_type=jnp.float32,
                precision=lax.Precision.HIGHEST)
    o_ref[...] = h * _dis_from(degs_ref)


def _tc_matmul_scale(x, w, degs):
    return pl.pallas_call(
        _mm_scale_body,
        out_shape=jax.ShapeDtypeStruct((x.shape[0], w.shape[1]), jnp.float32),
    )(x, w, degs)


RB = 2048


def _combine_body(s_any, h_ref, degs_ref, b_ref, wa_ref, wb_ref, o_ref,
                  s_v, sem):
    i = pl.program_id(0)
    pltpu.make_async_copy(s_any.at[:, :, pl.ds(i * RB, RB), :],
                          s_v, sem).start()
    dis = _dis_from(degs_ref)
    ha = h_ref[:, :DHALF]
    hb = h_ref[:, DHALF:]
    pltpu.make_async_copy(s_any.at[:, :, pl.ds(i * RB, RB), :],
                          s_v, sem).wait()
    s_ref = s_v
    ta = dis * (s_ref[0, 0] + s_ref[0, 1] + ha) + b_ref[:, :DHALF]
    tb = dis * (s_ref[1, 0] + s_ref[1, 1] + hb) + b_ref[:, DHALF:]
    ta = jnp.where(ta >= 0, ta, 0.01 * ta)
    tb = jnp.where(tb >= 0, tb, 0.01 * tb)
    o_ref[...] = (jnp.dot(ta, wa_ref[...],
                          preferred_element_type=jnp.float32,
                          precision=lax.Precision.HIGHEST)
                  + jnp.dot(tb, wb_ref[...],
                            preferred_element_type=jnp.float32,
                            precision=lax.Precision.HIGHEST)) * dis


def _tc_combine(s, h, degs, b, wa, wb):
    return pl.pallas_call(
        _combine_body,
        grid=(NACC // RB,),
        in_specs=[
            pl.BlockSpec(memory_space=pl.ANY),
            pl.BlockSpec((RB, D1P), lambda i: (i, 0)),
            pl.BlockSpec((2, RB), lambda i: (0, i)),
            pl.BlockSpec((1, D1P), lambda i: (0, 0)),
            pl.BlockSpec((DHALF, D2P), lambda i: (0, 0)),
            pl.BlockSpec((DHALF, D2P), lambda i: (0, 0)),
        ],
        out_specs=pl.BlockSpec((RB, D2P), lambda i: (i, 0)),
        out_shape=jax.ShapeDtypeStruct((NACC, D2P), jnp.float32),
        scratch_shapes=[pltpu.VMEM((2, 2, RB, DHALF), jnp.float32),
                        pltpu.SemaphoreType.DMA],
    )(s, h, degs, b, wa, wb)


RBF = 2000


def _final_body(s_any, h_ref, degst_ref, b_ref, o_ref, s_v, sem):
    i = pl.program_id(0)
    pltpu.make_async_copy(s_any.at[:, :, pl.ds(i * RBF, RBF), :],
                          s_v, sem).start()
    deg = degst_ref[:, 0] + degst_ref[:, 1] + 1.0
    dis = lax.rsqrt(deg)[:, None]
    pltpu.make_async_copy(s_any.at[:, :, pl.ds(i * RBF, RBF), :],
                          s_v, sem).wait()
    s = s_v[0, 0] + s_v[0, 1] + h_ref[...]
    t = dis * s + b_ref[...]
    valid = lax.broadcasted_iota(jnp.int32, (RBF, D2P), 1) < D2
    t = jnp.where(valid, t, -1e30)
    m = jnp.max(t, axis=1, keepdims=True)
    e = jnp.where(valid, jnp.exp(t - m), 0.0)
    se = jnp.sum(e, axis=1, keepdims=True)
    o_ref[...] = (t - m - jnp.log(se))[:, :D2]


def _tc_final(s, h, degst, b):
    return pl.pallas_call(
        _final_body,
        grid=(N // RBF,),
        in_specs=[
            pl.BlockSpec(memory_space=pl.ANY),
            pl.BlockSpec((RBF, D2P), lambda i: (i, 0)),
            pl.BlockSpec((RBF, 2), lambda i: (i, 0)),
            pl.BlockSpec((1, D2P), lambda i: (0, 0)),
        ],
        out_specs=pl.BlockSpec((RBF, D2), lambda i: (i, 0)),
        out_shape=jax.ShapeDtypeStruct((N, D2), jnp.float32),
        scratch_shapes=[pltpu.VMEM((1, 2, RBF, D2P), jnp.float32),
                        pltpu.SemaphoreType.DMA],
    )(s, h, degst, b)


def kernel(x, W1, b1, W2, b2, edge_index):
    src_p = edge_index[0].astype(jnp.int32).reshape(NW, NCHUNK, CHUNK)
    dst_p = edge_index[1].astype(jnp.int32).reshape(NW, NCHUNK, CHUNK)

    xp = jnp.pad(x, ((0, NACC - N), (0, 0)))
    zeros1 = jnp.zeros((NACC,), jnp.float32)
    zeros40 = jnp.zeros((NACC, DHALF), jnp.float32)
    zeros48 = jnp.zeros((NACC, D2P), jnp.float32)

    W1p = jnp.pad(W1, ((0, 0), (0, D1P - D1)))
    b1p = jnp.pad(b1, (0, D1P - D1)).reshape(1, D1P)
    W2p = jnp.pad(W2, ((0, D1P - D1), (0, D2P - D2)))
    b2p = jnp.pad(b2, (0, D2P - D2)).reshape(1, D2P)

    degs = _sc_degree(dst_p, zeros1)
    h1p = _tc_matmul_scale(xp, W1p, degs)
    s1 = _sc_aggregate(h1p, src_p, dst_p, zeros40, DHALF, 2)
    h2p = _tc_combine(s1, h1p, degs, b1p,
                      W2p[:DHALF], W2p[DHALF:])
    s2 = _sc_aggregate(h2p, src_p, dst_p, zeros48, D2P, 1)
    return _tc_final(s2, h2p, degs.T, b2p)


# speedup vs baseline: 1.0652x; 1.0652x over previous
import functools

import jax
import jax.numpy as jnp
from jax import lax
from jax.experimental import pallas as pl
from jax.experimental.pallas import tpu as pltpu
from jax.experimental.pallas import tpu_sc as plsc

N = 10000
E = 320000
NC = 2
NS = 16
NW = NC * NS
CHUNK = 125
NCHUNK = 80

NACC = 10240
ROWS_PER = NACC // NS

D1 = 67
D1P = 80
DHALF = D1P // 2
D2 = 40
D2P = 48


def _sc_degree(dst_hbm, zeros_hbm):
    mesh = plsc.VectorSubcoreMesh(core_axis_name="c", subcore_axis_name="s")

    @functools.partial(
        pl.kernel,
        out_type=jax.ShapeDtypeStruct((NC, NACC), jnp.float32),
        mesh=mesh,
        compiler_params=pltpu.CompilerParams(use_tc_tiling_on_sc=False),
        scratch_types=[
            pltpu.VMEM((NCHUNK, CHUNK), jnp.int32),
            pltpu.VMEM((128,), jnp.float32),
            pltpu.VMEM_SHARED((NACC,), jnp.float32),
        ],
    )
    def deg_kernel(dst_ref, zeros_ref, out_ref, dst_v, ones_v, acc_sh):
        cid = lax.axis_index("c")
        sid = lax.axis_index("s")
        wid = cid * NS + sid

        pltpu.sync_copy(zeros_ref.at[pl.ds(sid * ROWS_PER, ROWS_PER)],
                        acc_sh.at[pl.ds(sid * ROWS_PER, ROWS_PER)])
        pltpu.sync_copy(dst_ref.at[wid], dst_v)
        for i in range(8):
            ones_v[pl.ds(16 * i, 16)] = jnp.ones((16,), jnp.float32)
        plsc.subcore_barrier()

        def body(j, _):
            pltpu.sync_copy(ones_v.at[pl.ds(0, CHUNK)],
                            acc_sh.at[dst_v.at[j]], add=True)
            return ()

        lax.fori_loop(0, NCHUNK, body, (), unroll=False)
        plsc.subcore_barrier()
        pltpu.sync_copy(acc_sh.at[pl.ds(sid * ROWS_PER, ROWS_PER)],
                        out_ref.at[cid].at[pl.ds(sid * ROWS_PER, ROWS_PER)])

    return deg_kernel(dst_hbm, zeros_hbm)


def _sc_aggregate(h_hbm, src_hbm, dst_hbm, zeros_hbm, d, nhalf):
    mesh = plsc.VectorSubcoreMesh(core_axis_name="c", subcore_axis_name="s")

    @functools.partial(
        pl.kernel,
        out_type=jax.ShapeDtypeStruct((nhalf, NC, NACC, d), jnp.float32),
        mesh=mesh,
        compiler_params=pltpu.CompilerParams(use_tc_tiling_on_sc=False),
        scratch_types=[
            pltpu.VMEM((NCHUNK, CHUNK), jnp.int32),
            pltpu.VMEM((NCHUNK, CHUNK), jnp.int32),
            pltpu.VMEM((4, CHUNK, d), jnp.float32),
            pltpu.VMEM_SHARED((NACC, d), jnp.float32),
            pltpu.VMEM_SHARED((NACC, d), jnp.float32),
        ] + [pltpu.SemaphoreType.DMA] * 8,
    )
    def agg_kernel(h_ref, src_ref, dst_ref, zeros_ref, out_ref,
                   src_v, dst_v, rows_v, tbl_sh, acc_sh, *sems):
        cid = lax.axis_index("c")
        sid = lax.axis_index("s")
        wid = cid * NS + sid
        sl = pl.ds(sid * ROWS_PER, ROWS_PER)
        gsem = sems[:4]
        ssem = sems[4:]

        pltpu.sync_copy(src_ref.at[wid], src_v)
        pltpu.sync_copy(dst_ref.at[wid], dst_v)

        for half in range(nhalf):
            pltpu.sync_copy(h_ref.at[sl, pl.ds(half * d, d)], tbl_sh.at[sl])
            pltpu.sync_copy(zeros_ref.at[sl], acc_sh.at[sl])
            plsc.subcore_barrier()

            for k in range(2):
                pltpu.async_copy(tbl_sh.at[src_v.at[k]], rows_v.at[k],
                                 gsem[k])

            @pl.loop(0, NCHUNK, step=4)
            def _(j):
                for b in range(4):
                    k = j + b
                    buf = rows_v.at[b]
                    pltpu.make_async_copy(tbl_sh.at[src_v.at[k]],
                                          buf, gsem[b]).wait()
                    pltpu.async_copy(buf, acc_sh.at[dst_v.at[k]],
                                     ssem[b], add=True)

                    nb = (b + 2) % 4
                    nxt = rows_v.at[nb]

                    @pl.when(jnp.logical_and(k >= 2, k + 2 < NCHUNK))
                    def _():
                        pltpu.make_async_copy(nxt, acc_sh.at[dst_v.at[k]],
                                              ssem[nb]).wait()
                        pltpu.async_copy(tbl_sh.at[src_v.at[k + 2]],
                                         nxt, gsem[nb])

                    @pl.when(k < 2)
                    def _():
                        pltpu.async_copy(tbl_sh.at[src_v.at[k + 2]],
                                         nxt, gsem[nb])

            for b in range(4):
                pltpu.make_async_copy(rows_v.at[b],
                                      acc_sh.at[dst_v.at[0]],
                                      ssem[b]).wait()

            plsc.subcore_barrier()
            pltpu.sync_copy(acc_sh.at[sl],
                            out_ref.at[half].at[cid].at[sl])

    return agg_kernel(h_hbm, src_hbm, dst_hbm, zeros_hbm)


def _dis_from(degs_ref):
    deg = degs_ref[0, :] + degs_ref[1, :] + 1.0
    return lax.rsqrt(deg)[:, None]


def _mm_scale_body(x_ref, w_ref, degs_ref, o_ref):
    h = jnp.dot(x_ref[...], w_ref[...],
                preferred_element_type=jnp.float32,
                precision=lax.Precision.HIGHEST)
    o_ref[...] = h * _dis_from(degs_ref)


def _tc_matmul_scale(x, w, degs):
    return pl.pallas_call(
        _mm_scale_body,
        out_shape=jax.ShapeDtypeStruct((x.shape[0], w.shape[1]), jnp.float32),
    )(x, w, degs)


RB = 2048


def _combine_body(s_ref, h_ref, degs_ref, b_ref, wa_ref, wb_ref, o_ref):
    dis = _dis_from(degs_ref)
    ha = h_ref[:, :DHALF]
    hb = h_ref[:, DHALF:]
    ta = dis * (s_ref[0, 0] + s_ref[0, 1] + ha) + b_ref[:, :DHALF]
    tb = dis * (s_ref[1, 0] + s_ref[1, 1] + hb) + b_ref[:, DHALF:]
    ta = jnp.where(ta >= 0, ta, 0.01 * ta)
    tb = jnp.where(tb >= 0, tb, 0.01 * tb)
    o_ref[...] = (jnp.dot(ta, wa_ref[...],
                          preferred_element_type=jnp.float32,
                          precision=lax.Precision.HIGHEST)
                  + jnp.dot(tb, wb_ref[...],
                            preferred_element_type=jnp.float32,
                            precision=lax.Precision.HIGHEST)) * dis


def _tc_combine(s, h, degs, b, wa, wb):
    return pl.pallas_call(
        _combine_body,
        grid=(NACC // RB,),
        in_specs=[
            pl.BlockSpec((2, 2, RB, DHALF), lambda i: (0, 0, i, 0)),
            pl.BlockSpec((RB, D1P), lambda i: (i, 0)),
            pl.BlockSpec((2, RB), lambda i: (0, i)),
            pl.BlockSpec((1, D1P), lambda i: (0, 0)),
            pl.BlockSpec((DHALF, D2P), lambda i: (0, 0)),
            pl.BlockSpec((DHALF, D2P), lambda i: (0, 0)),
        ],
        out_specs=pl.BlockSpec((RB, D2P), lambda i: (i, 0)),
        out_shape=jax.ShapeDtypeStruct((NACC, D2P), jnp.float32),
    )(s, h, degs, b, wa, wb)


RBF = 2000


def _final_body(s_ref, h_ref, degst_ref, b_ref, o_ref):
    deg = degst_ref[:, 0] + degst_ref[:, 1] + 1.0
    dis = lax.rsqrt(deg)[:, None]
    s = s_ref[0, 0] + s_ref[0, 1] + h_ref[...]
    t = dis * s + b_ref[...]
    valid = lax.broadcasted_iota(jnp.int32, (RBF, D2P), 1) < D2
    t = jnp.where(valid, t, -1e30)
    m = jnp.max(t, axis=1, keepdims=True)
    e = jnp.where(valid, jnp.exp(t - m), 0.0)
    se = jnp.sum(e, axis=1, keepdims=True)
    o_ref[...] = (t - m - jnp.log(se))[:, :D2]


def _tc_final(s, h, degst, b):
    return pl.pallas_call(
        _final_body,
        grid=(N // RBF,),
        in_specs=[
            pl.BlockSpec((1, 2, RBF, D2P), lambda i: (0, 0, i, 0)),
            pl.BlockSpec((RBF, D2P), lambda i: (i, 0)),
            pl.BlockSpec((RBF, 2), lambda i: (i, 0)),
            pl.BlockSpec((1, D2P), lambda i: (0, 0)),
        ],
        out_specs=pl.BlockSpec((RBF, D2), lambda i: (i, 0)),
        out_shape=jax.ShapeDtypeStruct((N, D2), jnp.float32),
    )(s, h, degst, b)


def kernel(x, W1, b1, W2, b2, edge_index):
    src_p = edge_index[0].astype(jnp.int32).reshape(NW, NCHUNK, CHUNK)
    dst_p = edge_index[1].astype(jnp.int32).reshape(NW, NCHUNK, CHUNK)

    xp = jnp.pad(x, ((0, NACC - N), (0, 0)))
    zeros1 = jnp.zeros((NACC,), jnp.float32)
    zeros40 = jnp.zeros((NACC, DHALF), jnp.float32)
    zeros48 = jnp.zeros((NACC, D2P), jnp.float32)

    W1p = jnp.pad(W1, ((0, 0), (0, D1P - D1)))
    b1p = jnp.pad(b1, (0, D1P - D1)).reshape(1, D1P)
    W2p = jnp.pad(W2, ((0, D1P - D1), (0, D2P - D2)))
    b2p = jnp.pad(b2, (0, D2P - D2)).reshape(1, D2P)

    degs = _sc_degree(dst_p, zeros1)
    h1p = _tc_matmul_scale(xp, W1p, degs)
    s1 = _sc_aggregate(h1p, src_p, dst_p, zeros40, DHALF, 2)
    h2p = _tc_combine(s1, h1p, degs, b1p,
                      W2p[:DHALF], W2p[DHALF:])
    s2 = _sc_aggregate(h2p, src_p, dst_p, zeros48, D2P, 1)
    return _tc_final(s2, h2p, degs.T, b2p)


# speedup vs baseline: 1.0838x; 1.0175x over previous
import functools

import jax
import jax.numpy as jnp
from jax import lax
from jax.experimental import pallas as pl
from jax.experimental.pallas import tpu as pltpu
from jax.experimental.pallas import tpu_sc as plsc

N = 10000
E = 320000
NC = 2
NS = 16
NW = NC * NS
CHUNK = 125
NCHUNK = 80

NACC = 10240
ROWS_PER = NACC // NS

D1 = 67
D1P = 80
DHALF = D1P // 2
D2 = 40
D2P = 48


def _sc_degree(dst_hbm, zeros_hbm):
    mesh = plsc.VectorSubcoreMesh(core_axis_name="c", subcore_axis_name="s")

    @functools.partial(
        pl.kernel,
        out_type=jax.ShapeDtypeStruct((NC, NACC), jnp.float32),
        mesh=mesh,
        compiler_params=pltpu.CompilerParams(use_tc_tiling_on_sc=False),
        scratch_types=[
            pltpu.VMEM((NCHUNK, CHUNK), jnp.int32),
            pltpu.VMEM((128,), jnp.float32),
            pltpu.VMEM_SHARED((NACC,), jnp.float32),
            pltpu.SemaphoreType.DMA,
        ],
    )
    def deg_kernel(dst_ref, zeros_ref, out_ref, dst_v, ones_v, acc_sh, sem):
        cid = lax.axis_index("c")
        sid = lax.axis_index("s")
        wid = cid * NS + sid

        pltpu.sync_copy(zeros_ref.at[pl.ds(sid * ROWS_PER, ROWS_PER)],
                        acc_sh.at[pl.ds(sid * ROWS_PER, ROWS_PER)])
        pltpu.sync_copy(dst_ref.at[wid], dst_v)
        for i in range(8):
            ones_v[pl.ds(16 * i, 16)] = jnp.ones((16,), jnp.float32)
        plsc.subcore_barrier()

        def body(j, _):
            pltpu.async_copy(ones_v.at[pl.ds(0, CHUNK)],
                             acc_sh.at[dst_v.at[j]], sem, add=True)
            return ()

        lax.fori_loop(0, NCHUNK, body, (), unroll=False)

        def drain(j, _):
            pltpu.make_async_copy(ones_v.at[pl.ds(0, CHUNK)],
                                  acc_sh.at[dst_v.at[j]], sem).wait()
            return ()

        lax.fori_loop(0, NCHUNK, drain, (), unroll=False)
        plsc.subcore_barrier()
        pltpu.sync_copy(acc_sh.at[pl.ds(sid * ROWS_PER, ROWS_PER)],
                        out_ref.at[cid].at[pl.ds(sid * ROWS_PER, ROWS_PER)])

    return deg_kernel(dst_hbm, zeros_hbm)


def _sc_aggregate(h_hbm, src_hbm, dst_hbm, zeros_hbm, d, nhalf):
    mesh = plsc.VectorSubcoreMesh(core_axis_name="c", subcore_axis_name="s")

    @functools.partial(
        pl.kernel,
        out_type=jax.ShapeDtypeStruct((nhalf, NC, NACC, d), jnp.float32),
        mesh=mesh,
        compiler_params=pltpu.CompilerParams(use_tc_tiling_on_sc=False),
        scratch_types=[
            pltpu.VMEM((NCHUNK, CHUNK), jnp.int32),
            pltpu.VMEM((NCHUNK, CHUNK), jnp.int32),
            pltpu.VMEM((4, CHUNK, d), jnp.float32),
            pltpu.VMEM_SHARED((NACC, d), jnp.float32),
            pltpu.VMEM_SHARED((NACC, d), jnp.float32),
        ] + [pltpu.SemaphoreType.DMA] * 8,
    )
    def agg_kernel(h_ref, src_ref, dst_ref, zeros_ref, out_ref,
                   src_v, dst_v, rows_v, tbl_sh, acc_sh, *sems):
        cid = lax.axis_index("c")
        sid = lax.axis_index("s")
        wid = cid * NS + sid
        sl = pl.ds(sid * ROWS_PER, ROWS_PER)
        gsem = sems[:4]
        ssem = sems[4:]

        pltpu.sync_copy(src_ref.at[wid], src_v)
        pltpu.sync_copy(dst_ref.at[wid], dst_v)

        for half in range(nhalf):
            pltpu.sync_copy(h_ref.at[sl, pl.ds(half * d, d)], tbl_sh.at[sl])
            pltpu.sync_copy(zeros_ref.at[sl], acc_sh.at[sl])
            plsc.subcore_barrier()

            for k in range(2):
                pltpu.async_copy(tbl_sh.at[src_v.at[k]], rows_v.at[k],
                                 gsem[k])

            @pl.loop(0, NCHUNK, step=4)
            def _(j):
                for b in range(4):
                    k = j + b
                    buf = rows_v.at[b]
                    pltpu.make_async_copy(tbl_sh.at[src_v.at[k]],
                                          buf, gsem[b]).wait()
                    pltpu.async_copy(buf, acc_sh.at[dst_v.at[k]],
                                     ssem[b], add=True)

                    nb = (b + 2) % 4
                    nxt = rows_v.at[nb]

                    @pl.when(jnp.logical_and(k >= 2, k + 2 < NCHUNK))
                    def _():
                        pltpu.make_async_copy(nxt, acc_sh.at[dst_v.at[k]],
                                              ssem[nb]).wait()
                        pltpu.async_copy(tbl_sh.at[src_v.at[k + 2]],
                                         nxt, gsem[nb])

                    @pl.when(k < 2)
                    def _():
                        pltpu.async_copy(tbl_sh.at[src_v.at[k + 2]],
                                         nxt, gsem[nb])

            for b in range(4):
                pltpu.make_async_copy(rows_v.at[b],
                                      acc_sh.at[dst_v.at[0]],
                                      ssem[b]).wait()

            plsc.subcore_barrier()
            pltpu.sync_copy(acc_sh.at[sl],
                            out_ref.at[half].at[cid].at[sl])

    return agg_kernel(h_hbm, src_hbm, dst_hbm, zeros_hbm)


def _dis_from(degs_ref):
    deg = degs_ref[0, :] + degs_ref[1, :] + 1.0
    return lax.rsqrt(deg)[:, None]


def _mm_scale_body(x_ref, w_ref, degs_ref, o_ref):
    h = jnp.dot(x_ref[...], w_ref[...],
                preferred_element_type=jnp.float32,
                precision=lax.Precision.HIGHEST)
    o_ref[...] = h * _dis_from(degs_ref)


def _tc_matmul_scale(x, w, degs):
    return pl.pallas_call(
        _mm_scale_body,
        out_shape=jax.ShapeDtypeStruct((x.shape[0], w.shape[1]), jnp.float32),
    )(x, w, degs)


RB = 2048


def _combine_body(s_ref, h_ref, degs_ref, b_ref, wa_ref, wb_ref, o_ref):
    dis = _dis_from(degs_ref)
    ha = h_ref[:, :DHALF]
    hb = h_ref[:, DHALF:]
    ta = dis * (s_ref[0, 0] + s_ref[0, 1] + ha) + b_ref[:, :DHALF]
    tb = dis * (s_ref[1, 0] + s_ref[1, 1] + hb) + b_ref[:, DHALF:]
    ta = jnp.where(ta >= 0, ta, 0.01 * ta)
    tb = jnp.where(tb >= 0, tb, 0.01 * tb)
    o_ref[...] = (jnp.dot(ta, wa_ref[...],
                          preferred_element_type=jnp.float32,
                          precision=lax.Precision.HIGHEST)
                  + jnp.dot(tb, wb_ref[...],
                            preferred_element_type=jnp.float32,
                            precision=lax.Precision.HIGHEST)) * dis


def _tc_combine(s, h, degs, b, wa, wb):
    return pl.pallas_call(
        _combine_body,
        grid=(NACC // RB,),
        in_specs=[
            pl.BlockSpec((2, 2, RB, DHALF), lambda i: (0, 0, i, 0)),
            pl.BlockSpec((RB, D1P), lambda i: (i, 0)),
            pl.BlockSpec((2, RB), lambda i: (0, i)),
            pl.BlockSpec((1, D1P), lambda i: (0, 0)),
            pl.BlockSpec((DHALF, D2P), lambda i: (0, 0)),
            pl.BlockSpec((DHALF, D2P), lambda i: (0, 0)),
        ],
        out_specs=pl.BlockSpec((RB, D2P), lambda i: (i, 0)),
        out_shape=jax.ShapeDtypeStruct((NACC, D2P), jnp.float32),
    )(s, h, degs, b, wa, wb)


RBF = 2000


def _final_body(s_ref, h_ref, degst_ref, b_ref, o_ref):
    deg = degst_ref[:, 0] + degst_ref[:, 1] + 1.0
    dis = lax.rsqrt(deg)[:, None]
    s = s_ref[0, 0] + s_ref[0, 1] + h_ref[...]
    t = dis * s + b_ref[...]
    valid = lax.broadcasted_iota(jnp.int32, (RBF, D2P), 1) < D2
    t = jnp.where(valid, t, -1e30)
    m = jnp.max(t, axis=1, keepdims=True)
    e = jnp.where(valid, jnp.exp(t - m), 0.0)
    se = jnp.sum(e, axis=1, keepdims=True)
    o_ref[...] = (t - m - jnp.log(se))[:, :D2]


def _tc_final(s, h, degst, b):
    return pl.pallas_call(
        _final_body,
        grid=(N // RBF,),
        in_specs=[
            pl.BlockSpec((1, 2, RBF, D2P), lambda i: (0, 0, i, 0)),
            pl.BlockSpec((RBF, D2P), lambda i: (i, 0)),
            pl.BlockSpec((RBF, 2), lambda i: (i, 0)),
            pl.BlockSpec((1, D2P), lambda i: (0, 0)),
        ],
        out_specs=pl.BlockSpec((RBF, D2), lambda i: (i, 0)),
        out_shape=jax.ShapeDtypeStruct((N, D2), jnp.float32),
    )(s, h, degst, b)


def kernel(x, W1, b1, W2, b2, edge_index):
    src_p = edge_index[0].astype(jnp.int32).reshape(NW, NCHUNK, CHUNK)
    dst_p = edge_index[1].astype(jnp.int32).reshape(NW, NCHUNK, CHUNK)

    xp = jnp.pad(x, ((0, NACC - N), (0, 0)))
    zeros1 = jnp.zeros((NACC,), jnp.float32)
    zeros40 = jnp.zeros((NACC, DHALF), jnp.float32)
    zeros48 = jnp.zeros((NACC, D2P), jnp.float32)

    W1p = jnp.pad(W1, ((0, 0), (0, D1P - D1)))
    b1p = jnp.pad(b1, (0, D1P - D1)).reshape(1, D1P)
    W2p = jnp.pad(W2, ((0, D1P - D1), (0, D2P - D2)))
    b2p = jnp.pad(b2, (0, D2P - D2)).reshape(1, D2P)

    degs = _sc_degree(dst_p, zeros1)
    h1p = _tc_matmul_scale(xp, W1p, degs)
    s1 = _sc_aggregate(h1p, src_p, dst_p, zeros40, DHALF, 2)
    h2p = _tc_combine(s1, h1p, degs, b1p,
                      W2p[:DHALF], W2p[DHALF:])
    s2 = _sc_aggregate(h2p, src_p, dst_p, zeros48, D2P, 1)
    return _tc_final(s2, h2p, degs.T, b2p)
